# baseline (speedup 1.0000x reference)
import jax
import jax.numpy as jnp
from jax import lax
from jax.experimental import pallas as pl
from jax.experimental.pallas import tpu as pltpu
from jax.experimental.pallas import tpu_sc as plsc

N_DIMS = 64
SEQ_LEN = 200
BATCH = 4096

_INFO = plsc.get_sparse_core_info()
_NC = _INFO.num_cores
_NS = _INFO.num_subcores
_NW = _NC * _NS

_BATCH_PER_W = BATCH // _NW
_SPLIT = 104
_NBUF = 4


def _body(ex_hbm, cat_hbm, etab_hbm, ctab_hbm, ptab_hbm, out_hbm,
          p_v, ie0, ie1, ie2, ie3, ic0, ic1, ic2, ic3,
          bo0, bo1, bo2, bo3,
          si0, si1, si2, si3, se0, se1, se2, se3,
          sc0, sc1, sc2, sc3, so0, so1, so2, so3):
    ie = (ie0, ie1, ie2, ie3)
    ic = (ic0, ic1, ic2, ic3)
    bo = (bo0, bo1, bo2, bo3)
    si = (si0, si1, si2, si3)
    se = (se0, se1, se2, se3)
    sc = (sc0, sc1, sc2, sc3)
    so = (so0, so1, so2, so3)

    wid = lax.axis_index("s") * _NC + lax.axis_index("c")
    w_row = wid * _BATCH_PER_W

    pltpu.sync_copy(ptab_hbm, p_v)

    def issue_idx(ci, b):
        row = w_row + ci
        pltpu.async_copy(ex_hbm.at[row], ie[b], si[b])
        pltpu.async_copy(cat_hbm.at[row], ic[b], si[b])

    def wait_idx(ci, b):
        row = w_row + ci
        pltpu.make_async_copy(ex_hbm.at[row], ie[b], si[b]).wait()
        pltpu.make_async_copy(cat_hbm.at[row], ic[b], si[b]).wait()

    def issue_e(b):
        lo = pl.ds(0, _SPLIT)
        hi = pl.ds(_SPLIT, SEQ_LEN - _SPLIT)
        pltpu.async_copy(etab_hbm.at[ie[b].at[lo]], bo[b].at[lo], se[b])
        pltpu.async_copy(etab_hbm.at[ie[b].at[hi]], bo[b].at[hi], se[b])

    def wait_e(b):
        lo = pl.ds(0, _SPLIT)
        hi = pl.ds(_SPLIT, SEQ_LEN - _SPLIT)
        pltpu.make_async_copy(etab_hbm.at[ie[b].at[lo]], bo[b].at[lo], se[b]).wait()
        pltpu.make_async_copy(etab_hbm.at[ie[b].at[hi]], bo[b].at[hi], se[b]).wait()

    def issue_c(b):
        lo = pl.ds(0, _SPLIT)
        hi = pl.ds(_SPLIT, SEQ_LEN - _SPLIT)
        pltpu.async_copy(ctab_hbm.at[ic[b].at[lo]], bo[b].at[lo], sc[b], add=True)
        pltpu.async_copy(ctab_hbm.at[ic[b].at[hi]], bo[b].at[hi], sc[b], add=True)

    def wait_c(b):
        lo = pl.ds(0, _SPLIT)
        hi = pl.ds(_SPLIT, SEQ_LEN - _SPLIT)
        pltpu.make_async_copy(ctab_hbm.at[ic[b].at[lo]], bo[b].at[lo], sc[b]).wait()
        pltpu.make_async_copy(ctab_hbm.at[ic[b].at[hi]], bo[b].at[hi], sc[b]).wait()

    def wait_writeback(ci, b):
        row = w_row + ci
        pltpu.make_async_copy(bo[b], out_hbm.at[row], so[b]).wait()

    for b in range(_NBUF):
        issue_idx(b, b)
    for b in range(2):
        wait_idx(b, b)
        issue_e(b)

    def chunk(ci, b):
        wait_e(b)
        issue_c(b)

        fi = ci + 2
        fb = (b + 2) % _NBUF

        @pl.when(fi < _BATCH_PER_W)
        def _():
            @pl.when(fi >= _NBUF)
            def _():
                wait_writeback(fi - _NBUF, fb)
            wait_idx(fi, fb)
            issue_e(fb)

        wait_c(b)

        @pl.when(ci + _NBUF < _BATCH_PER_W)
        def _():
            issue_idx(ci + _NBUF, b)

        def row_body(r, carry2):
            for d in range(N_DIMS // 16):
                sl = pl.ds(d * 16, 16)
                bo[b][r, sl] = bo[b][r, sl] + p_v[r, sl]
            return carry2

        lax.fori_loop(0, SEQ_LEN, row_body, 0, unroll=2)
        row = w_row + ci
        pltpu.async_copy(bo[b], out_hbm.at[row], so[b])

    def outer(g, carry):
        for b in range(_NBUF):
            chunk(g * _NBUF + b, b)
        return carry

    lax.fori_loop(0, _BATCH_PER_W // _NBUF, outer, 0)

    for k in range(_NBUF):
        ci = _BATCH_PER_W - _NBUF + k
        wait_writeback(ci, ci % _NBUF)


@jax.jit
def _run(ex, cat, etab, ctab, ptab):
    mesh = plsc.VectorSubcoreMesh(core_axis_name="c", subcore_axis_name="s")
    f = pl.kernel(
        _body,
        out_type=jax.ShapeDtypeStruct((BATCH, SEQ_LEN, N_DIMS), jnp.float32),
        mesh=mesh,
        scratch_types=(
            [pltpu.VMEM((SEQ_LEN, N_DIMS), jnp.float32)]
            + [pltpu.VMEM((SEQ_LEN,), jnp.int32)] * (2 * _NBUF)
            + [pltpu.VMEM((SEQ_LEN, N_DIMS), jnp.float32)] * _NBUF
            + [pltpu.SemaphoreType.DMA] * (4 * _NBUF)
        ),
        compiler_params=pltpu.CompilerParams(use_tc_tiling_on_sc=False),
    )
    return f(ex, cat, etab, ctab, ptab)


def kernel(exercises, categories, exercise_table, category_table, position_table):
    return _run(exercises.astype(jnp.int32), categories.astype(jnp.int32),
                exercise_table, category_table, position_table)
